# baseline (speedup 1.0000x reference)
import functools
import math

import jax
import jax.numpy as jnp
from jax.experimental import pallas as pl

STUDENT_TEMP = 0.1
TEACHER_TEMP = 0.07
_LN2 = math.log(2.0)
_C_TEACH = 1.0 / (TEACHER_TEMP * _LN2)
_C_STUD = 1.0 / (STUDENT_TEMP * _LN2)

_BM = 384
_BM_MID = 1536
_BM_LOSS = 512


def _dot_cols(a, b):
    return jax.lax.dot_general(a, b, (((1,), (1,)), ((), ())),
                               preferred_element_type=jnp.float32)


def _dot_rows(a, b):
    return jax.lax.dot_general(a, b, (((0,), (0,)), ((), ())),
                               preferred_element_type=jnp.float32)


def _rowsum_cast_kernel(bm, t_ref, mt_ref, rowsum_ref, e16_ref):
    i = pl.program_id(0)

    @pl.when(i == 0)
    def _():
        rowsum_ref[...] = jnp.zeros_like(rowsum_ref)

    e = jnp.exp2(t_ref[...] * mt_ref[...])
    e16_ref[...] = e.astype(jnp.bfloat16)
    rowsum_ref[...] += _dot_rows(jnp.ones((bm, 1), jnp.float32), e)


def _mid_kernel(k_dim, e16_ref, rowsum1_ref, bn_ref, out2_ref, out3_ref):
    p = pl.program_id(0)
    i = pl.program_id(1)

    @pl.when((p == 0) & (i == 0))
    def _():
        out2_ref[...] = jnp.zeros_like(out2_ref)
        out3_ref[...] = jnp.zeros_like(out3_ref)

    rs1 = rowsum1_ref[...]
    r = jnp.where(p == 0,
                  jnp.sum(rs1) / (k_dim * rs1),
                  1.0 / (k_dim * out2_ref[...]))
    e = e16_ref[...].astype(jnp.float32)
    colsum = _dot_cols(e, r)
    c = 1.0 / (bn_ref[0, 0] * colsum)
    contrib = _dot_rows(c, e)

    @pl.when(p == 0)
    def _():
        out2_ref[...] += contrib

    @pl.when(p == 1)
    def _():
        out3_ref[...] += contrib


def _loss_kernel(k_dim, e16_ref, s_ref, rowsum_ref, ms_ref, w_ref, out_ref):
    i = pl.program_id(0)

    @pl.when(i == 0)
    def _():
        out_ref[...] = jnp.zeros_like(out_ref)

    r3 = 1.0 / (float(k_dim) * rowsum_ref[...])
    e = e16_ref[...].astype(jnp.float32)
    a = _dot_cols(e, r3)
    s2 = s_ref[...] * ms_ref[...]
    sumexp = _dot_cols(jnp.exp2(s2), jnp.ones((1, int(k_dim)), jnp.float32))
    lse = jnp.log(sumexp)
    dot = _dot_cols(e * s2, r3)
    loss = lse - (_LN2 * dot) / a
    out_ref[...] += jnp.sum(loss * w_ref[...]).reshape(1, 1)


def kernel(student_patch_tokens_masked, teacher_patch_tokens_masked,
           student_masks_flat, n_masked_patches_tensor):
    B, N, D = student_patch_tokens_masked.shape
    M = B * N
    t = teacher_patch_tokens_masked.reshape(M, D)
    s = student_patch_tokens_masked.reshape(M, D)

    mask_f = student_masks_flat.astype(jnp.float32)
    n_per_sample = jnp.clip(jnp.sum(mask_f, axis=-1), 1.0, None)
    w = (mask_f * (1.0 / n_per_sample)[:, None]).reshape(M, 1) * (1.0 / M)
    mask_col = mask_f.reshape(M, 1)
    mt = mask_col * _C_TEACH
    ms = mask_col * _C_STUD
    bn = n_masked_patches_tensor.astype(jnp.float32).reshape(1, 1)

    f32 = jnp.float32
    vec_spec = pl.BlockSpec((1, D), lambda i: (0, 0))
    col_spec = lambda bm: pl.BlockSpec((bm, 1), lambda i: (i, 0))
    big_spec = lambda bm: pl.BlockSpec((bm, D), lambda i: (i, 0))
    one_spec = pl.BlockSpec((1, 1), lambda i: (0, 0))
    vec_out = jax.ShapeDtypeStruct((1, D), f32)

    rowsum1, e16 = pl.pallas_call(
        functools.partial(_rowsum_cast_kernel, _BM),
        grid=(M // _BM,),
        in_specs=[big_spec(_BM), col_spec(_BM)],
        out_specs=[vec_spec, big_spec(_BM)],
        out_shape=[vec_out, jax.ShapeDtypeStruct((M, D), jnp.bfloat16)],
    )(t, mt)

    _, rowsum3 = pl.pallas_call(
        functools.partial(_mid_kernel, float(D)),
        grid=(2, M // _BM_MID),
        in_specs=[pl.BlockSpec((_BM_MID, D), lambda p, i: (i, 0)),
                  pl.BlockSpec((1, D), lambda p, i: (0, 0)),
                  pl.BlockSpec((1, 1), lambda p, i: (0, 0))],
        out_specs=[pl.BlockSpec((1, D), lambda p, i: (0, 0)),
                   pl.BlockSpec((1, D), lambda p, i: (0, 0))],
        out_shape=[vec_out, vec_out],
    )(e16, rowsum1, bn)

    loss = pl.pallas_call(
        functools.partial(_loss_kernel, D, ),
        grid=(M // _BM_LOSS,),
        in_specs=[big_spec(_BM_LOSS), big_spec(_BM_LOSS), vec_spec,
                  col_spec(_BM_LOSS), col_spec(_BM_LOSS)],
        out_specs=one_spec,
        out_shape=jax.ShapeDtypeStruct((1, 1), f32),
    )(e16, s, rowsum3, ms, w)

    return loss[0, 0]


# speedup vs baseline: 1.0042x; 1.0042x over previous
import functools
import math

import jax
import jax.numpy as jnp
from jax.experimental import pallas as pl

STUDENT_TEMP = 0.1
TEACHER_TEMP = 0.07
_LN2 = math.log(2.0)
_C_TEACH = 1.0 / (TEACHER_TEMP * _LN2)
_C_STUD = 1.0 / (STUDENT_TEMP * _LN2)

_BM = 384
_BM_MID = 1152
_BM_LOSS = 512


def _dot_cols(a, b):
    return jax.lax.dot_general(a, b, (((1,), (1,)), ((), ())),
                               preferred_element_type=jnp.float32)


def _dot_rows(a, b):
    return jax.lax.dot_general(a, b, (((0,), (0,)), ((), ())),
                               preferred_element_type=jnp.float32)


def _rowsum_cast_kernel(bm, t_ref, mt_ref, rowsum_ref, e16_ref):
    i = pl.program_id(0)

    @pl.when(i == 0)
    def _():
        rowsum_ref[...] = jnp.zeros_like(rowsum_ref)

    e = jnp.exp2(t_ref[...] * mt_ref[...])
    e16_ref[...] = e.astype(jnp.bfloat16)
    rowsum_ref[...] += _dot_rows(jnp.ones((bm, 1), jnp.float32), e)


def _mid_kernel(k_dim, e16_ref, rowsum1_ref, bn_ref, out2_ref, out3_ref):
    p = pl.program_id(0)
    i = pl.program_id(1)

    @pl.when((p == 0) & (i == 0))
    def _():
        out2_ref[...] = jnp.zeros_like(out2_ref)
        out3_ref[...] = jnp.zeros_like(out3_ref)

    rs1 = rowsum1_ref[...]
    r = jnp.where(p == 0,
                  jnp.sum(rs1) / (k_dim * rs1),
                  1.0 / (k_dim * out2_ref[...]))
    e = e16_ref[...].astype(jnp.float32)
    colsum = _dot_cols(e, r)
    c = 1.0 / (bn_ref[0, 0] * colsum)
    contrib = _dot_rows(c, e)

    @pl.when(p == 0)
    def _():
        out2_ref[...] += contrib

    @pl.when(p == 1)
    def _():
        out3_ref[...] += contrib


def _loss_kernel(k_dim, e16_ref, s_ref, rowsum_ref, ms_ref, w_ref, out_ref):
    i = pl.program_id(0)

    @pl.when(i == 0)
    def _():
        out_ref[...] = jnp.zeros_like(out_ref)

    r3 = 1.0 / (float(k_dim) * rowsum_ref[...])
    e = e16_ref[...].astype(jnp.float32)
    a = _dot_cols(e, r3)
    s2 = s_ref[...] * ms_ref[...]
    sumexp = _dot_cols(jnp.exp2(s2), jnp.ones((1, int(k_dim)), jnp.float32))
    lse = jnp.log(sumexp)
    dot = _dot_cols(e * s2, r3)
    loss = lse - (_LN2 * dot) / a
    out_ref[...] += jnp.sum(loss * w_ref[...]).reshape(1, 1)


def kernel(student_patch_tokens_masked, teacher_patch_tokens_masked,
           student_masks_flat, n_masked_patches_tensor):
    B, N, D = student_patch_tokens_masked.shape
    M = B * N
    t = teacher_patch_tokens_masked.reshape(M, D)
    s = student_patch_tokens_masked.reshape(M, D)

    mask_f = student_masks_flat.astype(jnp.float32)
    n_per_sample = jnp.clip(jnp.sum(mask_f, axis=-1), 1.0, None)
    w = (mask_f * (1.0 / n_per_sample)[:, None]).reshape(M, 1) * (1.0 / M)
    mask_col = mask_f.reshape(M, 1)
    mt = mask_col * _C_TEACH
    ms = mask_col * _C_STUD
    bn = n_masked_patches_tensor.astype(jnp.float32).reshape(1, 1)

    f32 = jnp.float32
    vec_spec = pl.BlockSpec((1, D), lambda i: (0, 0))
    col_spec = lambda bm: pl.BlockSpec((bm, 1), lambda i: (i, 0))
    big_spec = lambda bm: pl.BlockSpec((bm, D), lambda i: (i, 0))
    one_spec = pl.BlockSpec((1, 1), lambda i: (0, 0))
    vec_out = jax.ShapeDtypeStruct((1, D), f32)

    rowsum1, e16 = pl.pallas_call(
        functools.partial(_rowsum_cast_kernel, _BM),
        grid=(M // _BM,),
        in_specs=[big_spec(_BM), col_spec(_BM)],
        out_specs=[vec_spec, big_spec(_BM)],
        out_shape=[vec_out, jax.ShapeDtypeStruct((M, D), jnp.bfloat16)],
    )(t, mt)

    _, rowsum3 = pl.pallas_call(
        functools.partial(_mid_kernel, float(D)),
        grid=(2, M // _BM_MID),
        in_specs=[pl.BlockSpec((_BM_MID, D), lambda p, i: (i, 0)),
                  pl.BlockSpec((1, D), lambda p, i: (0, 0)),
                  pl.BlockSpec((1, 1), lambda p, i: (0, 0))],
        out_specs=[pl.BlockSpec((1, D), lambda p, i: (0, 0)),
                   pl.BlockSpec((1, D), lambda p, i: (0, 0))],
        out_shape=[vec_out, vec_out],
    )(e16, rowsum1, bn)

    loss = pl.pallas_call(
        functools.partial(_loss_kernel, D),
        grid=(M // _BM_LOSS,),
        in_specs=[big_spec(_BM_LOSS), big_spec(_BM_LOSS), vec_spec,
                  col_spec(_BM_LOSS), col_spec(_BM_LOSS)],
        out_specs=one_spec,
        out_shape=jax.ShapeDtypeStruct((1, 1), f32),
    )(e16, s, rowsum3, ms, w)

    return loss[0, 0]
